# baseline (speedup 1.0000x reference)
import functools

import jax
import jax.numpy as jnp
from jax import lax
from jax.experimental import pallas as pl
from jax.experimental.pallas import tpu as pltpu
from jax.experimental.pallas import tpu_sc as plsc

EMB = 64
BATCH = 16384
NUM_CORES = 2
NUM_SUBCORES = 16
NUM_WORKERS = NUM_CORES * NUM_SUBCORES
B_PER_W = BATCH // NUM_WORKERS
LANES = 16


@functools.partial(
    pl.kernel,
    out_type=jax.ShapeDtypeStruct((BATCH, EMB), jnp.float32),
    mesh=plsc.VectorSubcoreMesh(core_axis_name="c", subcore_axis_name="s"),
    compiler_params=pltpu.CompilerParams(use_tc_tiling_on_sc=False),
    scratch_types=[
        pltpu.VMEM((B_PER_W,), jnp.int32),
        pltpu.VMEM((B_PER_W,), jnp.int32),
        pltpu.VMEM((B_PER_W, EMB), jnp.float32),
        pltpu.VMEM((B_PER_W, EMB), jnp.float32),
        pltpu.SemaphoreType.DMA,
        pltpu.SemaphoreType.DMA,
    ],
)
def _gmf_sc(uid_hbm, mid_hbm, ut_hbm, it_hbm, out_hbm,
            uidx, midx, ubuf, mbuf, sem_u, sem_m):
    wid = lax.axis_index("s") * NUM_CORES + lax.axis_index("c")
    base = wid * B_PER_W
    pltpu.sync_copy(uid_hbm.at[pl.ds(base, B_PER_W)], uidx)
    pltpu.sync_copy(mid_hbm.at[pl.ds(base, B_PER_W)], midx)

    cp_u = pltpu.make_async_copy(ut_hbm.at[uidx], ubuf, sem_u)
    cp_m = pltpu.make_async_copy(it_hbm.at[midx], mbuf, sem_m)
    cp_u.start()
    cp_m.start()
    cp_u.wait()
    cp_m.wait()

    def mul(b, carry):
        for c in range(EMB // LANES):
            sl = pl.ds(c * LANES, LANES)
            ubuf[b, sl] = ubuf[b, sl] * mbuf[b, sl]
        return carry

    lax.fori_loop(0, B_PER_W, mul, 0)

    pltpu.sync_copy(ubuf, out_hbm.at[pl.ds(base, B_PER_W)])


def kernel(user_ids, movie_ids, user_table, item_table):
    uid = user_ids.astype(jnp.int32)
    mid = movie_ids.astype(jnp.int32)
    return _gmf_sc(uid, mid, user_table, item_table)
